# baseline (speedup 1.0000x reference)
import jax
import jax.numpy as jnp
from jax import lax
from jax.experimental import pallas as pl
from jax.experimental.pallas import tpu as pltpu
from jax.experimental.pallas import tpu_sc as plsc

_NB = 64
_GRID = _NB * _NB * _NB
_B = 16
_P = 131072
_NC = 2
_NS = 16
_L = 16
_ROUNDS = _B // _NC
_PPT = _P // _NS
_FPT = _PPT * 3
_NVEC = _FPT // _L
_GSLICE = _GRID // _NS
_CHUNK = 128
_NCHUNK = _PPT // _CHUNK
_RING = 4


def _body(x_hbm, out_hbm, pts, idx0, idx1, idx2, idx3, ones, zeros,
          maxv, allmax, shared, sem):
    idxs = (idx0, idx1, idx2, idx3)
    c = lax.axis_index("c")
    s = lax.axis_index("s")
    lane = lax.iota(jnp.int32, _L)
    czero = jnp.zeros((_L,), jnp.int32)
    cone = czero + 1
    ctwo = czero + 2

    for k in range(_CHUNK // _L):
        ones[pl.ds(k * _L, _L)] = jnp.ones((_L,), jnp.float32)

    def zero_body(i, _):
        zeros[pl.ds(i * _L, _L)] = jnp.zeros((_L,), jnp.float32)
        return 0
    lax.fori_loop(0, _GSLICE // _L, zero_body, 0)

    pltpu.sync_copy(zeros, shared.at[pl.ds(s * _GSLICE, _GSLICE)])

    def round_body(r, _):
        b = c * _ROUNDS + r

        pltpu.sync_copy(x_hbm.at[b, pl.ds(s * _PPT, _PPT)], pts)

        def max_body(i, m):
            pid = lane + i * _L
            mx = jnp.abs(plsc.load_gather(pts, [pid, czero]))
            my = jnp.abs(plsc.load_gather(pts, [pid, cone]))
            mz = jnp.abs(plsc.load_gather(pts, [pid, ctwo]))
            return jnp.maximum(m, jnp.maximum(mx, jnp.maximum(my, mz)))
        m = lax.fori_loop(0, _PPT // _L, max_body,
                          jnp.zeros((_L,), jnp.float32))
        maxv[...] = m
        pltpu.sync_copy(maxv, shared.at[pl.ds(_GRID + s * _L, _L)])
        plsc.subcore_barrier()

        pltpu.sync_copy(shared.at[pl.ds(_GRID, _NS * _L)], allmax)

        def gmax_body(i, mm):
            return jnp.maximum(mm, allmax[pl.ds(i * _L, _L)])
        mm = lax.fori_loop(0, _NS, gmax_body, jnp.zeros((_L,), jnp.float32))
        gmax = mm[0]
        for i in range(1, _L):
            gmax = jnp.maximum(gmax, mm[i])
        thick = jnp.maximum(2.0 * gmax, 1e-5)

        def super_body(go, _):
            for j in range(_RING):
                g = go * _RING + j
                for v in range(_CHUNK // _L):
                    pid = lane + (g * _CHUNK + v * _L)
                    x = plsc.load_gather(pts, [pid, czero])
                    y = plsc.load_gather(pts, [pid, cone])
                    z = plsc.load_gather(pts, [pid, ctwo])
                    cx = ((x + gmax) / thick * 64.0).astype(jnp.int32)
                    cy = ((y + gmax) / thick * 64.0).astype(jnp.int32)
                    cz = ((z + gmax) / thick * 64.0).astype(jnp.int32)
                    cx = jnp.clip(cx, 0, _NB - 1)
                    cy = jnp.clip(cy, 0, _NB - 1)
                    cz = jnp.clip(cz, 0, _NB - 1)
                    flat = (cx * _NB + cy) * _NB + cz
                    idxs[j][pl.ds(v * _L, _L)] = flat
                pltpu.async_copy(ones, shared.at[idxs[j]], sem)
            for j in range(_RING):
                pltpu.make_async_copy(ones, shared.at[idxs[j]], sem).wait()
            return 0
        lax.fori_loop(0, _NCHUNK // _RING, super_body, 0)
        plsc.subcore_barrier()

        sl = pl.ds(s * _GSLICE, _GSLICE)
        pltpu.sync_copy(shared.at[sl], out_hbm.at[b, sl])
        pltpu.sync_copy(zeros, shared.at[sl])
        return 0

    lax.fori_loop(0, _ROUNDS, round_body, 0)


_occupancy = pl.kernel(
    _body,
    out_type=jax.ShapeDtypeStruct((_B, _GRID), jnp.float32),
    mesh=plsc.VectorSubcoreMesh(
        core_axis_name="c", subcore_axis_name="s",
        num_cores=_NC, num_subcores=_NS),
    compiler_params=pltpu.CompilerParams(
        needs_layout_passes=False, use_tc_tiling_on_sc=False),
    scratch_types=[
        pltpu.VMEM((_PPT, 3), jnp.float32),
        pltpu.VMEM((_CHUNK,), jnp.int32),
        pltpu.VMEM((_CHUNK,), jnp.int32),
        pltpu.VMEM((_CHUNK,), jnp.int32),
        pltpu.VMEM((_CHUNK,), jnp.int32),
        pltpu.VMEM((_CHUNK,), jnp.float32),
        pltpu.VMEM((_GSLICE,), jnp.float32),
        pltpu.VMEM((_L,), jnp.float32),
        pltpu.VMEM((_NS * _L,), jnp.float32),
        pltpu.VMEM_SHARED((_GRID + _NS * _L,), jnp.float32),
        pltpu.SemaphoreType.DMA,
    ],
)


def kernel(input):
    return _occupancy(input)


# speedup vs baseline: 1.0849x; 1.0849x over previous
import jax
import jax.numpy as jnp
from jax import lax
from jax.experimental import pallas as pl
from jax.experimental.pallas import tpu as pltpu
from jax.experimental.pallas import tpu_sc as plsc

_NB = 64
_GRID = _NB * _NB * _NB
_B = 16
_P = 131072
_NC = 2
_NS = 16
_L = 16
_ROUNDS = _B // _NC
_PPT = _P // _NS
_FPT = _PPT * 3
_NVEC = _FPT // _L
_GSLICE = _GRID // _NS
_CHUNK = 128
_NCHUNK = _PPT // _CHUNK
_RING = 4


def _body(x_hbm, out_hbm, pts, idx0, idx1, idx2, idx3, ones, zeros,
          maxv, allmax, shared, sem):
    idxs = (idx0, idx1, idx2, idx3)
    c = lax.axis_index("c")
    s = lax.axis_index("s")
    lane = lax.iota(jnp.int32, _L)

    for k in range(_CHUNK // _L):
        ones[pl.ds(k * _L, _L)] = jnp.ones((_L,), jnp.float32)

    def zero_body(i, _):
        zeros[pl.ds(i * _L, _L)] = jnp.zeros((_L,), jnp.float32)
        return 0
    lax.fori_loop(0, _GSLICE // _L, zero_body, 0)

    pltpu.sync_copy(zeros, shared.at[pl.ds(s * _GSLICE, _GSLICE)])

    def round_body(r, _):
        b = c * _ROUNDS + r

        pltpu.sync_copy(x_hbm.at[pl.ds((b * _NS + s) * _FPT, _FPT)], pts)

        def max_body(i, m):
            v = pts[pl.ds(i * _L, _L)]
            return jnp.maximum(m, jnp.abs(v))
        m = lax.fori_loop(0, _NVEC, max_body, jnp.zeros((_L,), jnp.float32))
        maxv[...] = m
        pltpu.sync_copy(maxv, shared.at[pl.ds(_GRID + s * _L, _L)])
        plsc.subcore_barrier()

        pltpu.sync_copy(shared.at[pl.ds(_GRID, _NS * _L)], allmax)

        def gmax_body(i, mm):
            return jnp.maximum(mm, allmax[pl.ds(i * _L, _L)])
        mm = lax.fori_loop(0, _NS, gmax_body, jnp.zeros((_L,), jnp.float32))
        gmax = mm[0]
        for i in range(1, _L):
            gmax = jnp.maximum(gmax, mm[i])
        thick = jnp.maximum(2.0 * gmax, 1e-5)

        def super_body(go, _):
            for j in range(_RING):
                g = go * _RING + j
                for v in range(_CHUNK // _L):
                    pid = lane + (g * _CHUNK + v * _L)
                    fx = pid * 3
                    x = plsc.load_gather(pts, [fx])
                    y = plsc.load_gather(pts, [fx + 1])
                    z = plsc.load_gather(pts, [fx + 2])
                    cx = ((x + gmax) / thick * 64.0).astype(jnp.int32)
                    cy = ((y + gmax) / thick * 64.0).astype(jnp.int32)
                    cz = ((z + gmax) / thick * 64.0).astype(jnp.int32)
                    cx = jnp.clip(cx, 0, _NB - 1)
                    cy = jnp.clip(cy, 0, _NB - 1)
                    cz = jnp.clip(cz, 0, _NB - 1)
                    flat = (cx * _NB + cy) * _NB + cz
                    idxs[j][pl.ds(v * _L, _L)] = flat
                pltpu.async_copy(ones, shared.at[idxs[j]], sem)
            for j in range(_RING):
                pltpu.make_async_copy(ones, shared.at[idxs[j]], sem).wait()
            return 0
        lax.fori_loop(0, _NCHUNK // _RING, super_body, 0)
        plsc.subcore_barrier()

        sl = pl.ds(s * _GSLICE, _GSLICE)
        pltpu.sync_copy(shared.at[sl], out_hbm.at[b, sl])
        pltpu.sync_copy(zeros, shared.at[sl])
        return 0

    lax.fori_loop(0, _ROUNDS, round_body, 0)


_occupancy = pl.kernel(
    _body,
    out_type=jax.ShapeDtypeStruct((_B, _GRID), jnp.float32),
    mesh=plsc.VectorSubcoreMesh(
        core_axis_name="c", subcore_axis_name="s",
        num_cores=_NC, num_subcores=_NS),
    compiler_params=pltpu.CompilerParams(needs_layout_passes=False),
    scratch_types=[
        pltpu.VMEM((_FPT,), jnp.float32),
        pltpu.VMEM((_CHUNK,), jnp.int32),
        pltpu.VMEM((_CHUNK,), jnp.int32),
        pltpu.VMEM((_CHUNK,), jnp.int32),
        pltpu.VMEM((_CHUNK,), jnp.int32),
        pltpu.VMEM((_CHUNK,), jnp.float32),
        pltpu.VMEM((_GSLICE,), jnp.float32),
        pltpu.VMEM((_L,), jnp.float32),
        pltpu.VMEM((_NS * _L,), jnp.float32),
        pltpu.VMEM_SHARED((_GRID + _NS * _L,), jnp.float32),
        pltpu.SemaphoreType.DMA,
    ],
)


def kernel(input):
    return _occupancy(input.reshape(-1))


# speedup vs baseline: 27.6208x; 25.4583x over previous
import jax
import jax.numpy as jnp
from jax import lax
from jax.experimental import pallas as pl
from jax.experimental.pallas import tpu as pltpu
from jax.experimental.pallas import tpu_sc as plsc

_NB = 64
_GRID = _NB * _NB * _NB
_B = 16
_P = 131072
_NC = 2
_NS = 16
_L = 16
_ROUNDS = _B // _NC
_PPT = _P // _NS
_FPT = _PPT * 3
_NVEC = _FPT // _L
_GSLICE = _GRID // _NS
_CHUNK = 128
_NCHUNK = _PPT // _CHUNK
_RING = 4


def _body(x_hbm, out_hbm, pts, idx0, idx1, idx2, idx3, ones, zeros,
          maxv, allmax, shared, sem):
    idxs = (idx0, idx1, idx2, idx3)
    c = lax.axis_index("c")
    s = lax.axis_index("s")
    lane = lax.iota(jnp.int32, _L)

    for k in range(_CHUNK // _L):
        ones[pl.ds(k * _L, _L)] = jnp.ones((_L,), jnp.float32)

    def zero_body(i, _):
        zeros[pl.ds(i * _L, _L)] = jnp.zeros((_L,), jnp.float32)
        return 0
    lax.fori_loop(0, _GSLICE // _L, zero_body, 0)

    pltpu.sync_copy(zeros, shared.at[pl.ds(s * _GSLICE, _GSLICE)])

    def round_body(r, _):
        b = c * _ROUNDS + r

        pltpu.sync_copy(x_hbm.at[3 * b, pl.ds(s * _PPT, _PPT)],
                        pts.at[pl.ds(0, _PPT)])
        pltpu.sync_copy(x_hbm.at[3 * b + 1, pl.ds(s * _PPT, _PPT)],
                        pts.at[pl.ds(_PPT, _PPT)])
        pltpu.sync_copy(x_hbm.at[3 * b + 2, pl.ds(s * _PPT, _PPT)],
                        pts.at[pl.ds(2 * _PPT, _PPT)])

        def max_body(i, m):
            v = pts[pl.ds(i * _L, _L)]
            return jnp.maximum(m, jnp.abs(v))
        m = lax.fori_loop(0, _NVEC, max_body, jnp.zeros((_L,), jnp.float32))
        maxv[...] = m
        pltpu.sync_copy(maxv, shared.at[pl.ds(_GRID + s * _L, _L)])
        plsc.subcore_barrier()

        pltpu.sync_copy(shared.at[pl.ds(_GRID, _NS * _L)], allmax)

        def gmax_body(i, mm):
            return jnp.maximum(mm, allmax[pl.ds(i * _L, _L)])
        mm = lax.fori_loop(0, _NS, gmax_body, jnp.zeros((_L,), jnp.float32))
        gmax = mm[0]
        for i in range(1, _L):
            gmax = jnp.maximum(gmax, mm[i])
        thick = jnp.maximum(2.0 * gmax, 1e-5)

        def super_body(go, _):
            for j in range(_RING):
                g = go * _RING + j
                for v in range(_CHUNK // _L):
                    pb = g * _CHUNK + v * _L
                    x = pts[pl.ds(pb, _L)]
                    y = pts[pl.ds(_PPT + pb, _L)]
                    z = pts[pl.ds(2 * _PPT + pb, _L)]
                    cx = ((x + gmax) / thick * 64.0).astype(jnp.int32)
                    cy = ((y + gmax) / thick * 64.0).astype(jnp.int32)
                    cz = ((z + gmax) / thick * 64.0).astype(jnp.int32)
                    cx = jnp.clip(cx, 0, _NB - 1)
                    cy = jnp.clip(cy, 0, _NB - 1)
                    cz = jnp.clip(cz, 0, _NB - 1)
                    flat = (cx * _NB + cy) * _NB + cz
                    idxs[j][pl.ds(v * _L, _L)] = flat
                pltpu.async_copy(ones, shared.at[idxs[j]], sem)
            for j in range(_RING):
                pltpu.make_async_copy(ones, shared.at[idxs[j]], sem).wait()
            return 0
        lax.fori_loop(0, _NCHUNK // _RING, super_body, 0)
        plsc.subcore_barrier()

        sl = pl.ds(s * _GSLICE, _GSLICE)
        pltpu.sync_copy(shared.at[sl], out_hbm.at[b, sl])
        pltpu.sync_copy(zeros, shared.at[sl])
        return 0

    lax.fori_loop(0, _ROUNDS, round_body, 0)


_occupancy = pl.kernel(
    _body,
    out_type=jax.ShapeDtypeStruct((_B, _GRID), jnp.float32),
    mesh=plsc.VectorSubcoreMesh(
        core_axis_name="c", subcore_axis_name="s",
        num_cores=_NC, num_subcores=_NS),
    compiler_params=pltpu.CompilerParams(needs_layout_passes=False),
    scratch_types=[
        pltpu.VMEM((_FPT,), jnp.float32),
        pltpu.VMEM((_CHUNK,), jnp.int32),
        pltpu.VMEM((_CHUNK,), jnp.int32),
        pltpu.VMEM((_CHUNK,), jnp.int32),
        pltpu.VMEM((_CHUNK,), jnp.int32),
        pltpu.VMEM((_CHUNK,), jnp.float32),
        pltpu.VMEM((_GSLICE,), jnp.float32),
        pltpu.VMEM((_L,), jnp.float32),
        pltpu.VMEM((_NS * _L,), jnp.float32),
        pltpu.VMEM_SHARED((_GRID + _NS * _L,), jnp.float32),
        pltpu.SemaphoreType.DMA,
    ],
)


def kernel(input):
    return _occupancy(input.transpose(0, 2, 1).reshape(_B * 3, _P))


# speedup vs baseline: 35.7078x; 1.2928x over previous
import jax
import jax.numpy as jnp
from jax import lax
from jax.experimental import pallas as pl
from jax.experimental.pallas import tpu as pltpu
from jax.experimental.pallas import tpu_sc as plsc

_NB = 64
_GRID = _NB * _NB * _NB
_B = 16
_P = 131072
_NC = 2
_NS = 16
_L = 16
_ROUNDS = _B // _NC
_PPT = _P // _NS
_FPT = _PPT * 3
_NVEC = _FPT // _L
_GSLICE = _GRID // _NS
_CHUNK = 128
_NCHUNK = _PPT // _CHUNK
_RING = 4


def _body(x_hbm, out_hbm, pts, idx0, idx1, idx2, idx3, ones, zeros,
          maxv, allmax, shared, sem0, sem1, sem2, sem3, psem):
    idxs = (idx0, idx1, idx2, idx3)
    sems = (sem0, sem1, sem2, sem3)
    c = lax.axis_index("c")
    s = lax.axis_index("s")
    lane = lax.iota(jnp.int32, _L)

    for k in range(_CHUNK // _L):
        ones[pl.ds(k * _L, _L)] = jnp.ones((_L,), jnp.float32)

    def zero_body(i, _):
        zeros[pl.ds(i * _L, _L)] = jnp.zeros((_L,), jnp.float32)
        return 0
    lax.fori_loop(0, _GSLICE // _L, zero_body, 0)

    pltpu.sync_copy(zeros, shared.at[pl.ds(s * _GSLICE, _GSLICE)])

    def round_body(r, _):
        b = c * _ROUNDS + r

        for q in range(3):
            pltpu.async_copy(x_hbm.at[3 * b + q, pl.ds(s * _PPT, _PPT)],
                             pts.at[pl.ds(q * _PPT, _PPT)], psem)
        for q in range(3):
            pltpu.make_async_copy(x_hbm.at[3 * b + q, pl.ds(s * _PPT, _PPT)],
                                  pts.at[pl.ds(q * _PPT, _PPT)], psem).wait()

        def max_body(i, carry):
            m0, m1, m2, m3 = carry
            base = i * (4 * _L)
            v0 = pts[pl.ds(base, _L)]
            v1 = pts[pl.ds(base + _L, _L)]
            v2 = pts[pl.ds(base + 2 * _L, _L)]
            v3 = pts[pl.ds(base + 3 * _L, _L)]
            return (jnp.maximum(m0, jnp.abs(v0)),
                    jnp.maximum(m1, jnp.abs(v1)),
                    jnp.maximum(m2, jnp.abs(v2)),
                    jnp.maximum(m3, jnp.abs(v3)))
        z4 = jnp.zeros((_L,), jnp.float32)
        m0, m1, m2, m3 = lax.fori_loop(0, _NVEC // 4, max_body,
                                       (z4, z4, z4, z4), unroll=2)
        maxv[...] = jnp.maximum(jnp.maximum(m0, m1), jnp.maximum(m2, m3))
        pltpu.sync_copy(maxv, shared.at[pl.ds(_GRID + s * _L, _L)])
        plsc.subcore_barrier()

        pltpu.sync_copy(shared.at[pl.ds(_GRID, _NS * _L)], allmax)

        def gmax_body(i, mm):
            return jnp.maximum(mm, allmax[pl.ds(i * _L, _L)])
        mm = lax.fori_loop(0, _NS, gmax_body, jnp.zeros((_L,), jnp.float32))
        gmax = mm[0]
        for i in range(1, _L):
            gmax = jnp.maximum(gmax, mm[i])
        thick = jnp.maximum(2.0 * gmax, 1e-5)

        def compute_chunk(g, j):
            for v in range(_CHUNK // _L):
                pb = g * _CHUNK + v * _L
                x = pts[pl.ds(pb, _L)]
                y = pts[pl.ds(_PPT + pb, _L)]
                z = pts[pl.ds(2 * _PPT + pb, _L)]
                cx = ((x + gmax) / thick * 64.0).astype(jnp.int32)
                cy = ((y + gmax) / thick * 64.0).astype(jnp.int32)
                cz = ((z + gmax) / thick * 64.0).astype(jnp.int32)
                cx = jnp.clip(cx, 0, _NB - 1)
                cy = jnp.clip(cy, 0, _NB - 1)
                cz = jnp.clip(cz, 0, _NB - 1)
                flat = (cx * _NB + cy) * _NB + cz
                idxs[j][pl.ds(v * _L, _L)] = flat

        for j in range(_RING):
            compute_chunk(jnp.int32(j), j)
            pltpu.async_copy(ones, shared.at[idxs[j]], sems[j])

        def super_body(go, _):
            for j in range(_RING):
                pltpu.make_async_copy(ones, shared.at[idxs[j]],
                                      sems[j]).wait()
                compute_chunk(go * _RING + j, j)
                pltpu.async_copy(ones, shared.at[idxs[j]], sems[j])
            return 0
        lax.fori_loop(1, _NCHUNK // _RING, super_body, 0)
        for j in range(_RING):
            pltpu.make_async_copy(ones, shared.at[idxs[j]], sems[j]).wait()
        plsc.subcore_barrier()

        sl = pl.ds(s * _GSLICE, _GSLICE)
        pltpu.sync_copy(shared.at[sl], out_hbm.at[b, sl])
        pltpu.sync_copy(zeros, shared.at[sl])
        return 0

    lax.fori_loop(0, _ROUNDS, round_body, 0)


_occupancy = pl.kernel(
    _body,
    out_type=jax.ShapeDtypeStruct((_B, _GRID), jnp.float32),
    mesh=plsc.VectorSubcoreMesh(
        core_axis_name="c", subcore_axis_name="s",
        num_cores=_NC, num_subcores=_NS),
    compiler_params=pltpu.CompilerParams(needs_layout_passes=False),
    scratch_types=[
        pltpu.VMEM((_FPT,), jnp.float32),
        pltpu.VMEM((_CHUNK,), jnp.int32),
        pltpu.VMEM((_CHUNK,), jnp.int32),
        pltpu.VMEM((_CHUNK,), jnp.int32),
        pltpu.VMEM((_CHUNK,), jnp.int32),
        pltpu.VMEM((_CHUNK,), jnp.float32),
        pltpu.VMEM((_GSLICE,), jnp.float32),
        pltpu.VMEM((_L,), jnp.float32),
        pltpu.VMEM((_NS * _L,), jnp.float32),
        pltpu.VMEM_SHARED((_GRID + _NS * _L,), jnp.float32),
        pltpu.SemaphoreType.DMA,
        pltpu.SemaphoreType.DMA,
        pltpu.SemaphoreType.DMA,
        pltpu.SemaphoreType.DMA,
        pltpu.SemaphoreType.DMA,
    ],
)


def kernel(input):
    return _occupancy(input.transpose(0, 2, 1).reshape(_B * 3, _P))


# speedup vs baseline: 37.3557x; 1.0461x over previous
import jax
import jax.numpy as jnp
from jax import lax
from jax.experimental import pallas as pl
from jax.experimental.pallas import tpu as pltpu
from jax.experimental.pallas import tpu_sc as plsc

_NB = 64
_GRID = _NB * _NB * _NB
_B = 16
_P = 131072
_NC = 2
_NS = 16
_L = 16
_ROUNDS = _B // _NC
_PPT = _P // _NS
_FPT = _PPT * 3
_NVEC = _FPT // _L
_GSLICE = _GRID // _NS
_CHUNK = 128
_NCHUNK = _PPT // _CHUNK
_RING = 4


def _body(x_hbm, out_hbm, pts, idx0, idx1, idx2, idx3, ones, zeros,
          maxv, allmax, shared, sem0, sem1, sem2, sem3, psem):
    idxs = (idx0, idx1, idx2, idx3)
    sems = (sem0, sem1, sem2, sem3)
    c = lax.axis_index("c")
    s = lax.axis_index("s")
    lane = lax.iota(jnp.int32, _L)

    for k in range(_CHUNK // _L):
        ones[pl.ds(k * _L, _L)] = jnp.ones((_L,), jnp.float32)

    def zero_body(i, _):
        zeros[pl.ds(i * _L, _L)] = jnp.zeros((_L,), jnp.float32)
        return 0
    lax.fori_loop(0, _GSLICE // _L, zero_body, 0)

    pltpu.sync_copy(zeros, shared.at[pl.ds(s * _GSLICE, _GSLICE)])

    def round_body(r, _):
        b = c * _ROUNDS + r

        for q in range(3):
            pltpu.async_copy(
                x_hbm.at[pl.ds((3 * b + q) * _P + s * _PPT, _PPT)],
                pts.at[pl.ds(q * _PPT, _PPT)], psem)
        for q in range(3):
            pltpu.make_async_copy(
                x_hbm.at[pl.ds((3 * b + q) * _P + s * _PPT, _PPT)],
                pts.at[pl.ds(q * _PPT, _PPT)], psem).wait()

        def max_body(i, carry):
            m0, m1, m2, m3 = carry
            base = i * (4 * _L)
            v0 = pts[pl.ds(base, _L)]
            v1 = pts[pl.ds(base + _L, _L)]
            v2 = pts[pl.ds(base + 2 * _L, _L)]
            v3 = pts[pl.ds(base + 3 * _L, _L)]
            return (jnp.maximum(m0, jnp.abs(v0)),
                    jnp.maximum(m1, jnp.abs(v1)),
                    jnp.maximum(m2, jnp.abs(v2)),
                    jnp.maximum(m3, jnp.abs(v3)))
        z4 = jnp.zeros((_L,), jnp.float32)
        m0, m1, m2, m3 = lax.fori_loop(0, _NVEC // 4, max_body,
                                       (z4, z4, z4, z4), unroll=2)
        maxv[...] = jnp.maximum(jnp.maximum(m0, m1), jnp.maximum(m2, m3))
        pltpu.sync_copy(maxv, shared.at[pl.ds(_GRID + s * _L, _L)])
        plsc.subcore_barrier()

        pltpu.sync_copy(shared.at[pl.ds(_GRID, _NS * _L)], allmax)

        def gmax_body(i, mm):
            return jnp.maximum(mm, allmax[pl.ds(i * _L, _L)])
        mm = lax.fori_loop(0, _NS, gmax_body, jnp.zeros((_L,), jnp.float32))
        gmax = mm[0]
        for i in range(1, _L):
            gmax = jnp.maximum(gmax, mm[i])
        thick = jnp.maximum(2.0 * gmax, 1e-5)

        def compute_chunk(g, j):
            for v in range(_CHUNK // _L):
                pb = g * _CHUNK + v * _L
                x = pts[pl.ds(pb, _L)]
                y = pts[pl.ds(_PPT + pb, _L)]
                z = pts[pl.ds(2 * _PPT + pb, _L)]
                cx = ((x + gmax) / thick * 64.0).astype(jnp.int32)
                cy = ((y + gmax) / thick * 64.0).astype(jnp.int32)
                cz = ((z + gmax) / thick * 64.0).astype(jnp.int32)
                cx = jnp.clip(cx, 0, _NB - 1)
                cy = jnp.clip(cy, 0, _NB - 1)
                cz = jnp.clip(cz, 0, _NB - 1)
                flat = (cx * _NB + cy) * _NB + cz
                idxs[j][pl.ds(v * _L, _L)] = flat

        for j in range(_RING):
            compute_chunk(jnp.int32(j), j)
            pltpu.async_copy(ones, shared.at[idxs[j]], sems[j])

        def super_body(go, _):
            for j in range(_RING):
                pltpu.make_async_copy(ones, shared.at[idxs[j]],
                                      sems[j]).wait()
                compute_chunk(go * _RING + j, j)
                pltpu.async_copy(ones, shared.at[idxs[j]], sems[j])
            return 0
        lax.fori_loop(1, _NCHUNK // _RING, super_body, 0)
        for j in range(_RING):
            pltpu.make_async_copy(ones, shared.at[idxs[j]], sems[j]).wait()
        plsc.subcore_barrier()

        sl = pl.ds(s * _GSLICE, _GSLICE)
        pltpu.sync_copy(shared.at[sl], out_hbm.at[b, sl])
        pltpu.sync_copy(zeros, shared.at[sl])
        return 0

    lax.fori_loop(0, _ROUNDS, round_body, 0)


_occupancy = pl.kernel(
    _body,
    out_type=jax.ShapeDtypeStruct((_B, _GRID), jnp.float32),
    mesh=plsc.VectorSubcoreMesh(
        core_axis_name="c", subcore_axis_name="s",
        num_cores=_NC, num_subcores=_NS),
    compiler_params=pltpu.CompilerParams(needs_layout_passes=False),
    scratch_types=[
        pltpu.VMEM((_FPT,), jnp.float32),
        pltpu.VMEM((_CHUNK,), jnp.int32),
        pltpu.VMEM((_CHUNK,), jnp.int32),
        pltpu.VMEM((_CHUNK,), jnp.int32),
        pltpu.VMEM((_CHUNK,), jnp.int32),
        pltpu.VMEM((_CHUNK,), jnp.float32),
        pltpu.VMEM((_GSLICE,), jnp.float32),
        pltpu.VMEM((_L,), jnp.float32),
        pltpu.VMEM((_NS * _L,), jnp.float32),
        pltpu.VMEM_SHARED((_GRID + _NS * _L,), jnp.float32),
        pltpu.SemaphoreType.DMA,
        pltpu.SemaphoreType.DMA,
        pltpu.SemaphoreType.DMA,
        pltpu.SemaphoreType.DMA,
        pltpu.SemaphoreType.DMA,
    ],
)


def kernel(input):
    return _occupancy(input.transpose(0, 2, 1).reshape(-1))


# speedup vs baseline: 39.1224x; 1.0473x over previous
import jax
import jax.numpy as jnp
from jax import lax
from jax.experimental import pallas as pl
from jax.experimental.pallas import tpu as pltpu
from jax.experimental.pallas import tpu_sc as plsc

_NB = 64
_GRID = _NB * _NB * _NB
_B = 16
_P = 131072
_NC = 2
_NS = 16
_L = 16
_ROUNDS = _B // _NC
_PPT = _P // _NS
_FPT = _PPT * 3
_NVEC = _FPT // _L
_GSLICE = _GRID // _NS
_CHUNK = 128
_NCHUNK = _PPT // _CHUNK
_RING = 4


def _body(x_hbm, out_hbm, pts0, pts1, idx0, idx1, idx2, idx3, ones, zeros,
          maxv, allmax, shared, sem0, sem1, sem2, sem3, psem0, psem1):
    idxs = (idx0, idx1, idx2, idx3)
    sems = (sem0, sem1, sem2, sem3)
    c = lax.axis_index("c")
    s = lax.axis_index("s")

    for k in range(_CHUNK // _L):
        ones[pl.ds(k * _L, _L)] = jnp.ones((_L,), jnp.float32)

    def zero_body(i, _):
        zeros[pl.ds(i * _L, _L)] = jnp.zeros((_L,), jnp.float32)
        return 0
    lax.fori_loop(0, _GSLICE // _L, zero_body, 0)

    pltpu.sync_copy(zeros, shared.at[pl.ds(s * _GSLICE, _GSLICE)])

    def stage(r, ptsb, psemb):
        b = c * _ROUNDS + r
        for q in range(3):
            pltpu.async_copy(
                x_hbm.at[pl.ds((3 * b + q) * _P + s * _PPT, _PPT)],
                ptsb.at[pl.ds(q * _PPT, _PPT)], psemb)

    def do_round(r, ptsb, psemb):
        b = c * _ROUNDS + r
        for q in range(3):
            pltpu.make_async_copy(
                x_hbm.at[pl.ds((3 * b + q) * _P + s * _PPT, _PPT)],
                ptsb.at[pl.ds(q * _PPT, _PPT)], psemb).wait()

        def max_body(i, carry):
            m0, m1, m2, m3 = carry
            base = i * (4 * _L)
            v0 = ptsb[pl.ds(base, _L)]
            v1 = ptsb[pl.ds(base + _L, _L)]
            v2 = ptsb[pl.ds(base + 2 * _L, _L)]
            v3 = ptsb[pl.ds(base + 3 * _L, _L)]
            return (jnp.maximum(m0, jnp.abs(v0)),
                    jnp.maximum(m1, jnp.abs(v1)),
                    jnp.maximum(m2, jnp.abs(v2)),
                    jnp.maximum(m3, jnp.abs(v3)))
        z4 = jnp.zeros((_L,), jnp.float32)
        m0, m1, m2, m3 = lax.fori_loop(0, _NVEC // 4, max_body,
                                       (z4, z4, z4, z4), unroll=2)
        maxv[...] = jnp.maximum(jnp.maximum(m0, m1), jnp.maximum(m2, m3))
        pltpu.sync_copy(maxv, shared.at[pl.ds(_GRID + s * _L, _L)])
        plsc.subcore_barrier()

        pltpu.sync_copy(shared.at[pl.ds(_GRID, _NS * _L)], allmax)

        def gmax_body(i, mm):
            return jnp.maximum(mm, allmax[pl.ds(i * _L, _L)])
        mm = lax.fori_loop(0, _NS, gmax_body, jnp.zeros((_L,), jnp.float32))
        gmax = mm[0]
        for i in range(1, _L):
            gmax = jnp.maximum(gmax, mm[i])
        thick = jnp.maximum(2.0 * gmax, 1e-5)

        def compute_chunk(g, j):
            for v in range(_CHUNK // _L):
                pb = g * _CHUNK + v * _L
                x = ptsb[pl.ds(pb, _L)]
                y = ptsb[pl.ds(_PPT + pb, _L)]
                z = ptsb[pl.ds(2 * _PPT + pb, _L)]
                cx = ((x + gmax) / thick * 64.0).astype(jnp.int32)
                cy = ((y + gmax) / thick * 64.0).astype(jnp.int32)
                cz = ((z + gmax) / thick * 64.0).astype(jnp.int32)
                cx = jnp.clip(cx, 0, _NB - 1)
                cy = jnp.clip(cy, 0, _NB - 1)
                cz = jnp.clip(cz, 0, _NB - 1)
                flat = (cx * _NB + cy) * _NB + cz
                idxs[j][pl.ds(v * _L, _L)] = flat

        for j in range(_RING):
            compute_chunk(jnp.int32(j), j)
            pltpu.async_copy(ones, shared.at[idxs[j]], sems[j])

        def super_body(go, _):
            for j in range(_RING):
                pltpu.make_async_copy(ones, shared.at[idxs[j]],
                                      sems[j]).wait()
                compute_chunk(go * _RING + j, j)
                pltpu.async_copy(ones, shared.at[idxs[j]], sems[j])
            return 0
        lax.fori_loop(1, _NCHUNK // _RING, super_body, 0)
        for j in range(_RING):
            pltpu.make_async_copy(ones, shared.at[idxs[j]], sems[j]).wait()

        @pl.when(r < _ROUNDS - 2)
        def _():
            stage(r + 2, ptsb, psemb)

        plsc.subcore_barrier()

        sl = pl.ds(s * _GSLICE, _GSLICE)
        pltpu.sync_copy(shared.at[sl], out_hbm.at[b, sl])
        pltpu.sync_copy(zeros, shared.at[sl])

    stage(jnp.int32(0), pts0, psem0)
    stage(jnp.int32(1), pts1, psem1)

    def round2_body(k, _):
        do_round(2 * k, pts0, psem0)
        do_round(2 * k + 1, pts1, psem1)
        return 0
    lax.fori_loop(0, _ROUNDS // 2, round2_body, 0)


_occupancy = pl.kernel(
    _body,
    out_type=jax.ShapeDtypeStruct((_B, _GRID), jnp.float32),
    mesh=plsc.VectorSubcoreMesh(
        core_axis_name="c", subcore_axis_name="s",
        num_cores=_NC, num_subcores=_NS),
    compiler_params=pltpu.CompilerParams(needs_layout_passes=False),
    scratch_types=[
        pltpu.VMEM((_FPT,), jnp.float32),
        pltpu.VMEM((_FPT,), jnp.float32),
        pltpu.VMEM((_CHUNK,), jnp.int32),
        pltpu.VMEM((_CHUNK,), jnp.int32),
        pltpu.VMEM((_CHUNK,), jnp.int32),
        pltpu.VMEM((_CHUNK,), jnp.int32),
        pltpu.VMEM((_CHUNK,), jnp.float32),
        pltpu.VMEM((_GSLICE,), jnp.float32),
        pltpu.VMEM((_L,), jnp.float32),
        pltpu.VMEM((_NS * _L,), jnp.float32),
        pltpu.VMEM_SHARED((_GRID + _NS * _L,), jnp.float32),
        pltpu.SemaphoreType.DMA,
        pltpu.SemaphoreType.DMA,
        pltpu.SemaphoreType.DMA,
        pltpu.SemaphoreType.DMA,
        pltpu.SemaphoreType.DMA,
        pltpu.SemaphoreType.DMA,
    ],
)


def kernel(input):
    return _occupancy(input.transpose(0, 2, 1).reshape(-1))
